# baseline (speedup 1.0000x reference)
import functools
import math

import jax
import jax.numpy as jnp
from jax import lax
from jax.experimental import pallas as pl
from jax.experimental.pallas import tpu as pltpu
from jax.experimental.pallas import tpu_sc as plsc

D_MODEL = 64
SCALE = math.sqrt(D_MODEL)
NUM_CORES = 2
NUM_SUBCORES = 16
NUM_WORKERS = NUM_CORES * NUM_SUBCORES
CHUNK = 128
LANES = 16


@functools.lru_cache(maxsize=None)
def _build(n_rows: int):
    n_per_w = n_rows // NUM_WORKERS
    n_chunks = n_per_w // CHUNK
    assert n_per_w * NUM_WORKERS == n_rows and n_chunks * CHUNK == n_per_w

    mesh = plsc.VectorSubcoreMesh(
        core_axis_name="c", subcore_axis_name="s",
        num_cores=NUM_CORES, num_subcores=NUM_SUBCORES,
    )

    @functools.partial(
        pl.kernel,
        out_type=jax.ShapeDtypeStruct((n_rows, D_MODEL), jnp.float32),
        mesh=mesh,
        compiler_params=pltpu.CompilerParams(use_tc_tiling_on_sc=False),
        scratch_types=[
            pltpu.VMEM((CHUNK,), jnp.int32),
            pltpu.VMEM((CHUNK, D_MODEL), jnp.float32),
            pltpu.SemaphoreType.DMA,
        ],
    )
    def emb(idx_hbm, table_hbm, out_hbm, idx_v, rows_v, sem):
        wid = lax.axis_index("s") * NUM_CORES + lax.axis_index("c")
        base = wid * n_per_w

        def step(i, carry):
            off = base + i * CHUNK
            pltpu.sync_copy(idx_hbm.at[pl.ds(off, CHUNK)], idx_v)
            pltpu.async_copy(table_hbm.at[idx_v], rows_v, sem).wait()

            def scale_row(r, c):
                for j in range(D_MODEL // LANES):
                    sl = pl.ds(j * LANES, LANES)
                    rows_v[r, sl] = rows_v[r, sl] * SCALE
                return c

            lax.fori_loop(0, CHUNK, scale_row, 0)
            pltpu.sync_copy(rows_v, out_hbm.at[pl.ds(off, CHUNK)])
            return carry

        lax.fori_loop(0, n_chunks, step, 0)

    return emb


def kernel(token_id_tensor, embedding_table):
    b, s = token_id_tensor.shape
    n_rows = b * s
    idx = token_id_tensor.reshape(n_rows).astype(jnp.int32)
    out = _build(n_rows)(idx, embedding_table)
    return out.reshape(b, s, D_MODEL)


# speedup vs baseline: 1.2721x; 1.2721x over previous
import functools
import math

import jax
import jax.numpy as jnp
from jax import lax
from jax.experimental import pallas as pl
from jax.experimental.pallas import tpu as pltpu
from jax.experimental.pallas import tpu_sc as plsc

D_MODEL = 64
SCALE = math.sqrt(D_MODEL)
NUM_CORES = 2
NUM_SUBCORES = 16
NUM_WORKERS = NUM_CORES * NUM_SUBCORES
CHUNK = 128
LANES = 16
NBUF = 4


@functools.lru_cache(maxsize=None)
def _build(n_rows: int):
    n_per_w = n_rows // NUM_WORKERS
    n_chunks = n_per_w // CHUNK
    assert n_per_w * NUM_WORKERS == n_rows and n_chunks * CHUNK == n_per_w
    n_groups = n_chunks // NBUF
    assert n_groups * NBUF == n_chunks and n_groups >= 2

    mesh = plsc.VectorSubcoreMesh(
        core_axis_name="c", subcore_axis_name="s",
        num_cores=NUM_CORES, num_subcores=NUM_SUBCORES,
    )

    @functools.partial(
        pl.kernel,
        out_type=jax.ShapeDtypeStruct((n_rows, D_MODEL), jnp.float32),
        mesh=mesh,
        compiler_params=pltpu.CompilerParams(use_tc_tiling_on_sc=False),
        scratch_types=[
            pltpu.VMEM((n_chunks, CHUNK), jnp.int32),
            pltpu.VMEM((NBUF, CHUNK, D_MODEL), jnp.float32),
            pltpu.VMEM((NBUF, CHUNK, D_MODEL), jnp.float32),
            pltpu.SemaphoreType.DMA((NBUF,)),
            pltpu.SemaphoreType.DMA((NBUF,)),
        ],
    )
    def emb(idx_hbm, table_hbm, out_hbm, idx_v, ibuf, obuf, gsem, osem):
        wid = lax.axis_index("s") * NUM_CORES + lax.axis_index("c")
        row0 = wid * n_chunks
        base = wid * n_per_w

        pltpu.sync_copy(idx_hbm.at[pl.ds(row0, n_chunks)], idx_v)

        def start_gather(i, b):
            pltpu.async_copy(table_hbm.at[idx_v.at[i]], ibuf.at[b], gsem.at[b])

        def scale(b):
            def row(r, c):
                for j in range(D_MODEL // LANES):
                    sl = pl.ds(j * LANES, LANES)
                    obuf[b, r, sl] = ibuf[b, r, sl] * SCALE
                return c
            lax.fori_loop(0, CHUNK, row, 0)

        def start_write(i, b):
            pltpu.async_copy(obuf.at[b], out_hbm.at[pl.ds(base + i * CHUNK, CHUNK)],
                             osem.at[b])

        def wait_gather(i, b):
            pltpu.make_async_copy(table_hbm.at[idx_v.at[i]], ibuf.at[b],
                                  gsem.at[b]).wait()

        def wait_write(i, b):
            pltpu.make_async_copy(obuf.at[b], out_hbm.at[pl.ds(base + i * CHUNK, CHUNK)],
                                  osem.at[b]).wait()

        for b in range(NBUF):
            start_gather(b, b)

        for b in range(NBUF):
            wait_gather(b, b)
            scale(b)
            start_gather(b + NBUF, b)
            start_write(b, b)

        @pl.loop(1, n_groups - 1)
        def group(g):
            for b in range(NBUF):
                i = g * NBUF + b
                wait_gather(i, b)
                wait_write(i - NBUF, b)
                scale(b)
                start_gather(i + NBUF, b)
                start_write(i, b)

        for b in range(NBUF):
            i = (n_groups - 1) * NBUF + b
            wait_gather(i, b)
            wait_write(i - NBUF, b)
            scale(b)
            start_write(i, b)

        for b in range(NBUF):
            i = (n_groups - 1) * NBUF + b
            wait_write(i, b)

    return emb


def kernel(token_id_tensor, embedding_table):
    b, s = token_id_tensor.shape
    n_rows = b * s
    idx = token_id_tensor.reshape(n_rows // CHUNK, CHUNK).astype(jnp.int32)
    out = _build(n_rows)(idx, embedding_table)
    return out.reshape(b, s, D_MODEL)
